# Initial kernel scaffold; baseline (speedup 1.0000x reference)
import jax
import jax.numpy as jnp
from jax.experimental import pallas as pl


def kernel(x, params, edge_index):
    raise NotImplementedError("write your pallas kernel here")


# speedup vs baseline: 10.8447x; 10.8447x over previous
import functools

import jax
import jax.numpy as jnp
from jax import lax
from jax.experimental import pallas as pl
from jax.experimental.pallas import tpu as pltpu
from jax.experimental.pallas import tpu_sc as plsc

_T = 128
_L = 100
_DT = 1.0
_N = 10000
_N_PAD = 10112
_LANES = 16
_BLK = 128
_NW = 32


def _k_body(p_ref, k_ref):
    p = p_ref[...]
    sp = jnp.maximum(p, 0.0) + jnp.log1p(jnp.exp(-jnp.abs(p)))
    k_ref[...] = sp * 10.0 + 0.5


def _combine_body(p_ref, o_ref):
    o_ref[...] = (p_ref[0] + p_ref[1]).T


def _make_sc_route(n_blocks):
    mesh = plsc.VectorSubcoreMesh(core_axis_name="c", subcore_axis_name="s")
    rows_per_sub = _N_PAD // _LANES
    n_full = rows_per_sub // _BLK
    rem = rows_per_sub - n_full * _BLK

    @functools.partial(
        pl.kernel,
        mesh=mesh,
        compiler_params=pltpu.CompilerParams(needs_layout_passes=False),
        out_type=jax.ShapeDtypeStruct((2, _N_PAD, _T), jnp.float32),
        scratch_types=[
            pltpu.VMEM((_N_PAD,), jnp.float32),
            pltpu.VMEM((_BLK,), jnp.int32),
            pltpu.VMEM((_BLK,), jnp.int32),
            pltpu.VMEM((_BLK, _T), jnp.float32),
            pltpu.VMEM((_BLK, _T), jnp.float32),
            pltpu.VMEM_SHARED((_N_PAD, _T), jnp.float32),
            pltpu.SemaphoreType.DMA,
        ],
    )
    def sc_route(xt_hbm, k_hbm, src_hbm, dst_hbm, out_hbm,
                 k_v, src_v, dst_v, xrows, yrows, acc, sem):
        cid = lax.axis_index("c")
        sid = lax.axis_index("s")
        wid = sid * 2 + cid

        pltpu.sync_copy(k_hbm, k_v)

        lanes = lax.iota(jnp.int32, _LANES)
        zero16 = jnp.zeros((_LANES,), jnp.float32)

        def _zrow(i, carry):
            for c8 in range(_T // _LANES):
                xrows[i, pl.ds(c8 * _LANES, _LANES)] = zero16
            return carry

        lax.fori_loop(0, _BLK, _zrow, 0)
        base0 = sid * rows_per_sub
        for j in range(n_full):
            pltpu.sync_copy(xrows, acc.at[pl.ds(base0 + j * _BLK, _BLK)])
        if rem:
            pltpu.sync_copy(xrows.at[pl.ds(0, rem)],
                            acc.at[pl.ds(base0 + n_full * _BLK, rem)])
        plsc.subcore_barrier()

        def _block(b, carry):
            ebase = (wid * n_blocks + b) * _BLK
            pltpu.sync_copy(src_hbm.at[pl.ds(ebase, _BLK)], src_v)
            pltpu.sync_copy(dst_hbm.at[pl.ds(ebase, _BLK)], dst_v)
            pltpu.async_copy(xt_hbm.at[src_v], xrows, sem).wait()

            for g in range(_BLK // _LANES):
                rows = g * _LANES + lanes
                sg = src_v[pl.ds(g * _LANES, _LANES)]
                dg = dst_v[pl.ds(g * _LANES, _LANES)]
                ks = plsc.load_gather(k_v, [sg])
                kd = plsc.load_gather(k_v, [dg])
                a = (2.0 * _DT) / (ks + kd)
                r = jnp.exp(-a)
                r_l = jnp.exp(-float(_L) * a)
                norm = a * (1.0 - r_l) / (1.0 - r)
                c = a / (norm + 1e-8)
                cr_l = c * r_l

                def _t1(t, y):
                    tcol = jnp.full((_LANES,), t, jnp.int32)
                    xs = plsc.load_gather(xrows, [rows, tcol])
                    y = r * y + c * xs
                    plsc.store_scatter(yrows, [rows, tcol], y)
                    return y

                y = lax.fori_loop(0, _L, _t1, jnp.zeros((_LANES,), jnp.float32))

                def _t2(t, y):
                    tcol = jnp.full((_LANES,), t, jnp.int32)
                    ocol = jnp.full((_LANES,), t - _L, jnp.int32)
                    xs = plsc.load_gather(xrows, [rows, tcol])
                    xo = plsc.load_gather(xrows, [rows, ocol])
                    y = r * y + c * xs - cr_l * xo
                    plsc.store_scatter(yrows, [rows, tcol], y)
                    return y

                lax.fori_loop(_L, _T, _t2, y)

            pltpu.sync_copy(yrows, acc.at[dst_v], add=True)
            return carry

        lax.fori_loop(0, n_blocks, _block, 0)
        plsc.subcore_barrier()

        pltpu.sync_copy(acc.at[pl.ds(base0, rows_per_sub)],
                        out_hbm.at[cid, pl.ds(base0, rows_per_sub)])

    return sc_route


def kernel(x, params, edge_index):
    n = x.shape[1]
    e_raw = edge_index.shape[1]
    diag = jnp.arange(n, dtype=edge_index.dtype)
    src = jnp.concatenate([edge_index[0], diag])
    dst = jnp.concatenate([edge_index[1], diag])
    e_tot = e_raw + n
    e_pad = -(-e_tot // (_NW * _BLK)) * (_NW * _BLK)
    pad = e_pad - e_tot
    src = jnp.concatenate([src, jnp.zeros((pad,), jnp.int32)])
    dst = jnp.concatenate([dst, jnp.full((pad,), n, jnp.int32)])
    n_blocks = e_pad // (_NW * _BLK)

    p2d = jnp.pad(params, (0, _N_PAD - n)).reshape(_N_PAD // 128, 128)
    k2d = pl.pallas_call(
        _k_body,
        out_shape=jax.ShapeDtypeStruct((_N_PAD // 128, 128), jnp.float32),
    )(p2d)
    k_flat = k2d.reshape(_N_PAD)

    xt = x.T

    partials = _make_sc_route(n_blocks)(xt, k_flat, src, dst)

    out = pl.pallas_call(
        _combine_body,
        grid=(_N_PAD // 128,),
        in_specs=[pl.BlockSpec((2, 128, 128), lambda i: (0, i, 0))],
        out_specs=pl.BlockSpec((128, 128), lambda i: (0, i)),
        out_shape=jax.ShapeDtypeStruct((_T, _N_PAD), jnp.float32),
    )(partials)
    return out[:, :n]


# speedup vs baseline: 10.8518x; 1.0006x over previous
import functools

import jax
import jax.numpy as jnp
from jax import lax
from jax.experimental import pallas as pl
from jax.experimental.pallas import tpu as pltpu
from jax.experimental.pallas import tpu_sc as plsc

_T = 128
_L = 100
_DT = 1.0
_N = 10000
_N_PAD = 10112
_LANES = 16
_BLK = 128
_NW = 32
_U = 4
_UNROLL = 2


def _k_body(p_ref, k_ref):
    p = p_ref[...]
    sp = jnp.maximum(p, 0.0) + jnp.log1p(jnp.exp(-jnp.abs(p)))
    k_ref[...] = sp * 10.0 + 0.5


def _combine_body(p_ref, o_ref):
    o_ref[...] = (p_ref[0] + p_ref[1]).T


def _make_sc_route(n_blocks):
    mesh = plsc.VectorSubcoreMesh(core_axis_name="c", subcore_axis_name="s")
    rows_per_sub = _N_PAD // _LANES
    n_full = rows_per_sub // _BLK
    rem = rows_per_sub - n_full * _BLK

    @functools.partial(
        pl.kernel,
        mesh=mesh,
        compiler_params=pltpu.CompilerParams(needs_layout_passes=False),
        out_type=jax.ShapeDtypeStruct((2, _N_PAD, _T), jnp.float32),
        scratch_types=[
            pltpu.VMEM((_N_PAD,), jnp.float32),
            pltpu.VMEM((_BLK,), jnp.int32),
            pltpu.VMEM((_BLK,), jnp.int32),
            pltpu.VMEM((_BLK, _T), jnp.float32),
            pltpu.VMEM((_BLK, _T), jnp.float32),
            pltpu.VMEM_SHARED((_N_PAD, _T), jnp.float32),
            pltpu.SemaphoreType.DMA,
        ],
    )
    def sc_route(xt_hbm, k_hbm, src_hbm, dst_hbm, out_hbm,
                 k_v, src_v, dst_v, xrows, yrows, acc, sem):
        cid = lax.axis_index("c")
        sid = lax.axis_index("s")
        wid = sid * 2 + cid

        pltpu.sync_copy(k_hbm, k_v)

        lanes = lax.iota(jnp.int32, _LANES)
        zero16 = jnp.zeros((_LANES,), jnp.float32)

        def _zrow(i, carry):
            for c8 in range(_T // _LANES):
                xrows[i, pl.ds(c8 * _LANES, _LANES)] = zero16
            return carry

        lax.fori_loop(0, _BLK, _zrow, 0)
        base0 = sid * rows_per_sub
        for j in range(n_full):
            pltpu.sync_copy(xrows, acc.at[pl.ds(base0 + j * _BLK, _BLK)])
        if rem:
            pltpu.sync_copy(xrows.at[pl.ds(0, rem)],
                            acc.at[pl.ds(base0 + n_full * _BLK, rem)])
        plsc.subcore_barrier()

        def _block(b, carry):
            ebase = (wid * n_blocks + b) * _BLK
            pltpu.sync_copy(src_hbm.at[pl.ds(ebase, _BLK)], src_v)
            pltpu.sync_copy(dst_hbm.at[pl.ds(ebase, _BLK)], dst_v)
            pltpu.async_copy(xt_hbm.at[src_v], xrows, sem).wait()

            n_grp = _BLK // _LANES
            for g0 in range(0, n_grp, _U):
                rows_u, r_u, c_u, crl_u = [], [], [], []
                for u in range(_U):
                    g = g0 + u
                    rows_u.append(g * _LANES + lanes)
                    sg = src_v[pl.ds(g * _LANES, _LANES)]
                    dg = dst_v[pl.ds(g * _LANES, _LANES)]
                    ks = plsc.load_gather(k_v, [sg])
                    kd = plsc.load_gather(k_v, [dg])
                    a = (2.0 * _DT) / (ks + kd)
                    r = jnp.exp(-a)
                    r_l = jnp.exp(-float(_L) * a)
                    norm = a * (1.0 - r_l) / (1.0 - r)
                    c = a / (norm + 1e-8)
                    r_u.append(r)
                    c_u.append(c)
                    crl_u.append(c * r_l)

                def _t1(t, ys):
                    tcol = jnp.full((_LANES,), t, jnp.int32)
                    out = []
                    for u in range(_U):
                        xs = plsc.load_gather(xrows, [rows_u[u], tcol])
                        y = r_u[u] * ys[u] + c_u[u] * xs
                        plsc.store_scatter(yrows, [rows_u[u], tcol], y)
                        out.append(y)
                    return tuple(out)

                zeros = jnp.zeros((_LANES,), jnp.float32)
                ys = lax.fori_loop(0, _L, _t1, (zeros,) * _U,
                                   unroll=_UNROLL)

                def _t2(t, ys):
                    tcol = jnp.full((_LANES,), t, jnp.int32)
                    ocol = jnp.full((_LANES,), t - _L, jnp.int32)
                    out = []
                    for u in range(_U):
                        xs = plsc.load_gather(xrows, [rows_u[u], tcol])
                        xo = plsc.load_gather(xrows, [rows_u[u], ocol])
                        y = r_u[u] * ys[u] + c_u[u] * xs - crl_u[u] * xo
                        plsc.store_scatter(yrows, [rows_u[u], tcol], y)
                        out.append(y)
                    return tuple(out)

                lax.fori_loop(_L, _T, _t2, ys, unroll=_UNROLL)

            pltpu.sync_copy(yrows, acc.at[dst_v], add=True)
            return carry

        lax.fori_loop(0, n_blocks, _block, 0)
        plsc.subcore_barrier()

        pltpu.sync_copy(acc.at[pl.ds(base0, rows_per_sub)],
                        out_hbm.at[cid, pl.ds(base0, rows_per_sub)])

    return sc_route


def kernel(x, params, edge_index):
    n = x.shape[1]
    e_raw = edge_index.shape[1]
    diag = jnp.arange(n, dtype=edge_index.dtype)
    src = jnp.concatenate([edge_index[0], diag])
    dst = jnp.concatenate([edge_index[1], diag])
    e_tot = e_raw + n
    e_pad = -(-e_tot // (_NW * _BLK)) * (_NW * _BLK)
    pad = e_pad - e_tot
    src = jnp.concatenate([src, jnp.zeros((pad,), jnp.int32)])
    dst = jnp.concatenate([dst, jnp.full((pad,), n, jnp.int32)])
    n_blocks = e_pad // (_NW * _BLK)

    p2d = jnp.pad(params, (0, _N_PAD - n)).reshape(_N_PAD // 128, 128)
    k2d = pl.pallas_call(
        _k_body,
        out_shape=jax.ShapeDtypeStruct((_N_PAD // 128, 128), jnp.float32),
    )(p2d)
    k_flat = k2d.reshape(_N_PAD)

    xt = x.T

    partials = _make_sc_route(n_blocks)(xt, k_flat, src, dst)

    out = pl.pallas_call(
        _combine_body,
        grid=(_N_PAD // 128,),
        in_specs=[pl.BlockSpec((2, 128, 128), lambda i: (0, i, 0))],
        out_specs=pl.BlockSpec((128, 128), lambda i: (0, i)),
        out_shape=jax.ShapeDtypeStruct((_T, _N_PAD), jnp.float32),
    )(partials)
    return out[:, :n]


# speedup vs baseline: 12.4441x; 1.1467x over previous
import functools

import jax
import jax.numpy as jnp
from jax import lax
from jax.experimental import pallas as pl
from jax.experimental.pallas import tpu as pltpu
from jax.experimental.pallas import tpu_sc as plsc

_T = 128
_L = 100
_DT = 1.0
_N = 10000
_N_PAD = 10112
_LANES = 16
_BLK = 128
_NW = 32
_U = 4
_UNROLL = 2


def _k_body(p_ref, k_ref):
    p = p_ref[...]
    sp = jnp.maximum(p, 0.0) + jnp.log1p(jnp.exp(-jnp.abs(p)))
    k_ref[...] = sp * 10.0 + 0.5


def _combine_body(p_ref, o_ref):
    o_ref[...] = (p_ref[0] + p_ref[1]).T


def _make_sc_route(n_blocks):
    mesh = plsc.VectorSubcoreMesh(core_axis_name="c", subcore_axis_name="s")
    rows_per_sub = _N_PAD // _LANES
    n_full = rows_per_sub // _BLK
    rem = rows_per_sub - n_full * _BLK

    @functools.partial(
        pl.kernel,
        mesh=mesh,
        compiler_params=pltpu.CompilerParams(needs_layout_passes=False),
        out_type=jax.ShapeDtypeStruct((2, _N_PAD, _T), jnp.float32),
        scratch_types=[
            pltpu.VMEM((_N_PAD,), jnp.float32),
            pltpu.VMEM((_BLK,), jnp.int32),
            pltpu.VMEM((_BLK,), jnp.int32),
            pltpu.VMEM((_BLK, _T), jnp.float32),
            pltpu.VMEM((_BLK, _T), jnp.float32),
            pltpu.VMEM_SHARED((_N_PAD, _T), jnp.float32),
            pltpu.SemaphoreType.DMA,
        ],
    )
    def sc_route(xt_hbm, k_hbm, src_hbm, dst_hbm, out_hbm,
                 k_v, src_v, dst_v, xrows, yrows, acc, sem):
        cid = lax.axis_index("c")
        sid = lax.axis_index("s")
        wid = sid * 2 + cid

        pltpu.sync_copy(k_hbm, k_v)

        lanes = lax.iota(jnp.int32, _LANES)
        zero16 = jnp.zeros((_LANES,), jnp.float32)

        def _zrow(i, carry):
            for c8 in range(_T // _LANES):
                xrows[i, pl.ds(c8 * _LANES, _LANES)] = zero16
            return carry

        lax.fori_loop(0, _BLK, _zrow, 0)
        base0 = sid * rows_per_sub
        for j in range(n_full):
            pltpu.sync_copy(xrows, acc.at[pl.ds(base0 + j * _BLK, _BLK)])
        if rem:
            pltpu.sync_copy(xrows.at[pl.ds(0, rem)],
                            acc.at[pl.ds(base0 + n_full * _BLK, rem)])
        plsc.subcore_barrier()

        def _block(b, carry):
            ebase = (wid * n_blocks + b) * _BLK
            pltpu.sync_copy(src_hbm.at[pl.ds(ebase, _BLK)], src_v)
            pltpu.sync_copy(dst_hbm.at[pl.ds(ebase, _BLK)], dst_v)
            pltpu.async_copy(xt_hbm.at[src_v], xrows, sem).wait()

            n_grp = _BLK // _LANES
            for g0 in range(0, n_grp, _U):
                rows_u, r_u, c_u, crl_u = [], [], [], []
                for u in range(_U):
                    g = g0 + u
                    rows_u.append(g * _LANES + lanes)
                    sg = src_v[pl.ds(g * _LANES, _LANES)]
                    dg = dst_v[pl.ds(g * _LANES, _LANES)]
                    ks = plsc.load_gather(k_v, [sg])
                    kd = plsc.load_gather(k_v, [dg])
                    a = (2.0 * _DT) / (ks + kd)
                    r = jnp.exp(-a)
                    r_l = jnp.exp(-float(_L) * a)
                    norm = a * (1.0 - r_l) / (1.0 - r)
                    c = a / (norm + 1e-8)
                    r_u.append(r)
                    c_u.append(c)
                    crl_u.append(c * r_l)

                def _t1(i, ys):
                    t = 2 * i
                    ca = jnp.full((_LANES,), t, jnp.int32)
                    cb = jnp.full((_LANES,), t + 1, jnp.int32)
                    xa = [plsc.load_gather(xrows, [rows_u[u], ca])
                          for u in range(_U)]
                    xb = [plsc.load_gather(xrows, [rows_u[u], cb])
                          for u in range(_U)]
                    ya = [r_u[u] * ys[u] + c_u[u] * xa[u] for u in range(_U)]
                    yb = [r_u[u] * ya[u] + c_u[u] * xb[u] for u in range(_U)]
                    for u in range(_U):
                        plsc.store_scatter(yrows, [rows_u[u], ca], ya[u])
                    for u in range(_U):
                        plsc.store_scatter(yrows, [rows_u[u], cb], yb[u])
                    return tuple(yb)

                zeros = jnp.zeros((_LANES,), jnp.float32)
                ys = lax.fori_loop(0, _L // 2, _t1, (zeros,) * _U)

                def _t2(i, ys):
                    t = _L + 2 * i
                    ca = jnp.full((_LANES,), t, jnp.int32)
                    cb = jnp.full((_LANES,), t + 1, jnp.int32)
                    oa = jnp.full((_LANES,), t - _L, jnp.int32)
                    ob = jnp.full((_LANES,), t + 1 - _L, jnp.int32)
                    xa = [plsc.load_gather(xrows, [rows_u[u], ca])
                          for u in range(_U)]
                    xb = [plsc.load_gather(xrows, [rows_u[u], cb])
                          for u in range(_U)]
                    xoa = [plsc.load_gather(xrows, [rows_u[u], oa])
                           for u in range(_U)]
                    xob = [plsc.load_gather(xrows, [rows_u[u], ob])
                           for u in range(_U)]
                    ya = [r_u[u] * ys[u] + c_u[u] * xa[u] - crl_u[u] * xoa[u]
                          for u in range(_U)]
                    yb = [r_u[u] * ya[u] + c_u[u] * xb[u] - crl_u[u] * xob[u]
                          for u in range(_U)]
                    for u in range(_U):
                        plsc.store_scatter(yrows, [rows_u[u], ca], ya[u])
                    for u in range(_U):
                        plsc.store_scatter(yrows, [rows_u[u], cb], yb[u])
                    return tuple(yb)

                lax.fori_loop(0, (_T - _L) // 2, _t2, ys)

            pltpu.sync_copy(yrows, acc.at[dst_v], add=True)
            return carry

        lax.fori_loop(0, n_blocks, _block, 0)
        plsc.subcore_barrier()

        pltpu.sync_copy(acc.at[pl.ds(base0, rows_per_sub)],
                        out_hbm.at[cid, pl.ds(base0, rows_per_sub)])

    return sc_route


def kernel(x, params, edge_index):
    n = x.shape[1]
    e_raw = edge_index.shape[1]
    diag = jnp.arange(n, dtype=edge_index.dtype)
    src = jnp.concatenate([edge_index[0], diag])
    dst = jnp.concatenate([edge_index[1], diag])
    e_tot = e_raw + n
    e_pad = -(-e_tot // (_NW * _BLK)) * (_NW * _BLK)
    pad = e_pad - e_tot
    src = jnp.concatenate([src, jnp.zeros((pad,), jnp.int32)])
    dst = jnp.concatenate([dst, jnp.full((pad,), n, jnp.int32)])
    n_blocks = e_pad // (_NW * _BLK)

    p2d = jnp.pad(params, (0, _N_PAD - n)).reshape(_N_PAD // 128, 128)
    k2d = pl.pallas_call(
        _k_body,
        out_shape=jax.ShapeDtypeStruct((_N_PAD // 128, 128), jnp.float32),
    )(p2d)
    k_flat = k2d.reshape(_N_PAD)

    xt = x.T

    partials = _make_sc_route(n_blocks)(xt, k_flat, src, dst)

    out = pl.pallas_call(
        _combine_body,
        grid=(_N_PAD // 128,),
        in_specs=[pl.BlockSpec((2, 128, 128), lambda i: (0, i, 0))],
        out_specs=pl.BlockSpec((128, 128), lambda i: (0, i)),
        out_shape=jax.ShapeDtypeStruct((_T, _N_PAD), jnp.float32),
    )(partials)
    return out[:, :n]
